# Initial kernel scaffold; baseline (speedup 1.0000x reference)
import jax
import jax.numpy as jnp
from jax.experimental import pallas as pl


def kernel(feats, edge_index, W1, b1, W2, b2, cb1, cb2, p1w1, p1b1, p1w2, p1b2, p2w1, p2b1, p2w2, p2b2):
    raise NotImplementedError("write your pallas kernel here")


# speedup vs baseline: 4.2437x; 4.2437x over previous
import functools

import jax
import jax.numpy as jnp
from jax import lax
from jax.experimental import pallas as pl
from jax.experimental.pallas import tpu as pltpu
from jax.experimental.pallas import tpu_sc as plsc

N = 10000
NPAD = 10240
E = 320000
NW = 32
NS = 16
ET = E // NW
CHUNK = 80
NCHUNK = ET // CHUNK
ROWS_PER_TILE = NPAD // NS

@functools.lru_cache(maxsize=None)
def _mesh():
    return plsc.VectorSubcoreMesh(core_axis_name="c", subcore_axis_name="s",
                                  num_cores=2, num_subcores=NS)


def _deg_body(srcr, dstr, zeros1, out_hbm, src_v, dst_v, ones_v, acc_o, acc_i):
    c = lax.axis_index("c")
    s = lax.axis_index("s")
    wid = s * 2 + c
    pltpu.sync_copy(zeros1.at[pl.ds(s * ROWS_PER_TILE, ROWS_PER_TILE)],
                    acc_o.at[pl.ds(s * ROWS_PER_TILE, ROWS_PER_TILE)])
    pltpu.sync_copy(zeros1.at[pl.ds(s * ROWS_PER_TILE, ROWS_PER_TILE)],
                    acc_i.at[pl.ds(s * ROWS_PER_TILE, ROWS_PER_TILE)])
    for i in range(CHUNK // 16):
        ones_v[pl.ds(i * 16, 16)] = jnp.ones((16,), jnp.float32)
    pltpu.sync_copy(srcr.at[wid], src_v)
    pltpu.sync_copy(dstr.at[wid], dst_v)
    plsc.subcore_barrier()

    def step(j, carry):
        pltpu.sync_copy(ones_v, acc_o.at[src_v.at[j]], add=True)
        pltpu.sync_copy(ones_v, acc_i.at[dst_v.at[j]], add=True)
        return carry

    lax.fori_loop(0, NCHUNK, step, 0)
    plsc.subcore_barrier()
    pltpu.sync_copy(acc_o.at[pl.ds(s * ROWS_PER_TILE, ROWS_PER_TILE)],
                    out_hbm.at[0, c, pl.ds(s * ROWS_PER_TILE, ROWS_PER_TILE)])
    pltpu.sync_copy(acc_i.at[pl.ds(s * ROWS_PER_TILE, ROWS_PER_TILE)],
                    out_hbm.at[1, c, pl.ds(s * ROWS_PER_TILE, ROWS_PER_TILE)])


@functools.lru_cache(maxsize=None)
def _deg_kernel():
    return pl.kernel(
        _deg_body,
        out_type=jax.ShapeDtypeStruct((2, 2, NPAD), jnp.float32),
        mesh=_mesh(),
        scratch_types=[
            pltpu.VMEM((NCHUNK, CHUNK), jnp.int32),
            pltpu.VMEM((NCHUNK, CHUNK), jnp.int32),
            pltpu.VMEM((CHUNK,), jnp.float32),
            pltpu.VMEM_SHARED((NPAD,), jnp.float32),
            pltpu.VMEM_SHARED((NPAD,), jnp.float32),
        ],
    )


def _agg_body(table, srcr, dstr, zeros2, out_hbm,
              src_v, dst_v, rows_v, acc, sem):
    c = lax.axis_index("c")
    s = lax.axis_index("s")
    wid = s * 2 + c
    pltpu.sync_copy(zeros2.at[pl.ds(s * ROWS_PER_TILE, ROWS_PER_TILE)],
                    acc.at[pl.ds(s * ROWS_PER_TILE, ROWS_PER_TILE)])
    pltpu.sync_copy(srcr.at[wid], src_v)
    pltpu.sync_copy(dstr.at[wid], dst_v)
    plsc.subcore_barrier()

    def step(j, carry):
        pltpu.async_copy(table.at[src_v.at[j]], rows_v, sem).wait()
        pltpu.sync_copy(rows_v, acc.at[dst_v.at[j]], add=True)
        return carry

    lax.fori_loop(0, NCHUNK, step, 0)
    plsc.subcore_barrier()
    pltpu.sync_copy(acc.at[pl.ds(s * ROWS_PER_TILE, ROWS_PER_TILE)],
                    out_hbm.at[c, pl.ds(s * ROWS_PER_TILE, ROWS_PER_TILE)])


@functools.lru_cache(maxsize=None)
def _agg_kernel():
    return pl.kernel(
        _agg_body,
        out_type=jax.ShapeDtypeStruct((2, NPAD, 128), jnp.float32),
        mesh=_mesh(),
        scratch_types=[
            pltpu.VMEM((NCHUNK, CHUNK), jnp.int32),
            pltpu.VMEM((NCHUNK, CHUNK), jnp.int32),
            pltpu.VMEM((CHUNK, 128), jnp.float32),
            pltpu.VMEM_SHARED((NPAD, 128), jnp.float32),
            pltpu.SemaphoreType.DMA,
        ],
    )


def _proj_body(t1, t2, idx1r, idx2r, o1, o2, i1_v, i2_v, rows_v, sem):
    c = lax.axis_index("c")
    s = lax.axis_index("s")
    wid = s * 2 + c
    pltpu.sync_copy(idx1r.at[wid], i1_v)
    pltpu.sync_copy(idx2r.at[wid], i2_v)
    for j in range(4):
        pltpu.async_copy(t1.at[i1_v.at[j]], rows_v, sem).wait()
        pltpu.sync_copy(rows_v, o1.at[pl.ds(wid * 320 + j * CHUNK, CHUNK)])
        pltpu.async_copy(t2.at[i2_v.at[j]], rows_v, sem).wait()
        pltpu.sync_copy(rows_v, o2.at[pl.ds(wid * 320 + j * CHUNK, CHUNK)])


@functools.lru_cache(maxsize=None)
def _proj_kernel():
    return pl.kernel(
        _proj_body,
        out_type=[jax.ShapeDtypeStruct((NPAD, 128), jnp.float32),
                  jax.ShapeDtypeStruct((NPAD, 128), jnp.float32)],
        mesh=_mesh(),
        scratch_types=[
            pltpu.VMEM((4, CHUNK), jnp.int32),
            pltpu.VMEM((4, CHUNK), jnp.int32),
            pltpu.VMEM((CHUNK, 128), jnp.float32),
            pltpu.SemaphoreType.DMA,
        ],
    )


R = 1024


def _t0_body(degp_ref, feats_ref, xs_ref, rs_ref):
    d_out = degp_ref[0, 0] + degp_ref[0, 1]
    d_in = degp_ref[1, 0] + degp_ref[1, 1]
    rs_out = lax.rsqrt(jnp.maximum(d_out, 1.0))
    rs_in = lax.rsqrt(jnp.maximum(d_in, 1.0))
    xs_ref[...] = feats_ref[...] * rs_out[:, None]
    rs_ref[0, :] = rs_out
    rs_ref[1, :] = rs_in


def _t0(degp, feats):
    return pl.pallas_call(
        _t0_body,
        grid=(10,),
        in_specs=[
            pl.BlockSpec((2, 2, R), lambda i: (0, 0, i)),
            pl.BlockSpec((R, 128), lambda i: (i, 0)),
        ],
        out_specs=[
            pl.BlockSpec((R, 128), lambda i: (i, 0)),
            pl.BlockSpec((2, R), lambda i: (0, i)),
        ],
        out_shape=[jax.ShapeDtypeStruct((NPAD, 128), jnp.float32),
                   jax.ShapeDtypeStruct((2, NPAD), jnp.float32)],
    )(degp, feats)


def _t1_body(aggp_ref, rs_ref, w1_ref, b1_ref, ha_ref, hb_ref):
    a = (aggp_ref[0] + aggp_ref[1]) * rs_ref[1, :][:, None]
    h = jnp.dot(a, w1_ref[...], preferred_element_type=jnp.float32)
    h = jnp.maximum(h + b1_ref[...], 0.0) * rs_ref[0, :][:, None]
    ha_ref[...] = h[:, :128]
    hb_ref[...] = h[:, 128:]


def _t1(aggp, rs, W1, b1):
    return pl.pallas_call(
        _t1_body,
        grid=(10,),
        in_specs=[
            pl.BlockSpec((2, R, 128), lambda i: (0, i, 0)),
            pl.BlockSpec((2, R), lambda i: (0, i)),
            pl.BlockSpec((128, 256), lambda i: (0, 0)),
            pl.BlockSpec((1, 256), lambda i: (0, 0)),
        ],
        out_specs=[
            pl.BlockSpec((R, 128), lambda i: (i, 0)),
            pl.BlockSpec((R, 128), lambda i: (i, 0)),
        ],
        out_shape=[jax.ShapeDtypeStruct((NPAD, 128), jnp.float32),
                   jax.ShapeDtypeStruct((NPAD, 128), jnp.float32)],
    )(aggp, rs, W1, b1)


def _knorm_body(cb_ref, w1_ref, b1_ref, w2_ref, b2_ref, cbn_ref, tab_ref):
    cb = cb_ref[...]
    nrm = jnp.sqrt(jnp.sum(cb * cb, axis=1, keepdims=True))
    cbn = cb / (nrm + 1e-12)
    cbn_ref[...] = cbn
    t = jnp.dot(cbn, w1_ref[...], preferred_element_type=jnp.float32)
    t = jnp.maximum(t + b1_ref[...], 0.0)
    t = jnp.dot(t, w2_ref[...], preferred_element_type=jnp.float32)
    CB = t.shape[0]
    tab_ref[...] = jnp.concatenate(
        [t + b2_ref[...], jnp.zeros((CB, 64), jnp.float32)], axis=1)


def _knorm(cb, w1, b1, w2, b2):
    CB = cb.shape[0]
    return pl.pallas_call(
        _knorm_body,
        grid=(1,),
        in_specs=[
            pl.BlockSpec((CB, 256), lambda i: (0, 0)),
            pl.BlockSpec((256, 128), lambda i: (0, 0)),
            pl.BlockSpec((1, 128), lambda i: (0, 0)),
            pl.BlockSpec((128, 64), lambda i: (0, 0)),
            pl.BlockSpec((1, 64), lambda i: (0, 0)),
        ],
        out_specs=[
            pl.BlockSpec((CB, 256), lambda i: (0, 0)),
            pl.BlockSpec((CB, 128), lambda i: (0, 0)),
        ],
        out_shape=[jax.ShapeDtypeStruct((CB, 256), jnp.float32),
                   jax.ShapeDtypeStruct((CB, 128), jnp.float32)],
    )(cb, w1, b1, w2, b2)


def _argmax_rows(s):
    m = jnp.max(s, axis=1)
    ii = lax.broadcasted_iota(jnp.int32, s.shape, 1)
    idx = jnp.min(jnp.where(s == m[:, None], ii, jnp.int32(2**30)), axis=1)
    return m, idx


def _t2_body(aa_ref, ab_ref, rs_ref, w2_ref, b2_ref, c1t_ref, c2t_ref,
             i1_ref, i2_ref, sums_ref):
    i = pl.program_id(0)
    a = jnp.concatenate([aa_ref[0] + aa_ref[1], ab_ref[0] + ab_ref[1]],
                        axis=1) * rs_ref[1, :][:, None]
    h = jnp.dot(a, w2_ref[...], preferred_element_type=jnp.float32)
    h = h + b2_ref[...]
    nrm2 = jnp.sum(h * h, axis=1, keepdims=True)
    nrm = jnp.sqrt(nrm2) + 1e-12
    hn = h / nrm
    s1 = jnp.dot(hn, c1t_ref[...], preferred_element_type=jnp.float32)
    s2 = jnp.dot(hn, c2t_ref[...], preferred_element_type=jnp.float32)
    m1, idx1 = _argmax_rows(s1)
    m2, idx2 = _argmax_rows(s2)
    i1_ref[...] = idx1.reshape(1, 1, R)
    i2_ref[...] = idx2.reshape(1, 1, R)
    valid = (i * R + lax.broadcasted_iota(jnp.int32, (R, 1), 0)) < N
    p0 = jnp.sum(jnp.where(valid, nrm2, 0.0)).reshape(1, 1)
    p1 = jnp.sum(jnp.where(valid, m1[:, None] * nrm, 0.0)).reshape(1, 1)
    p2 = jnp.sum(jnp.where(valid, m2[:, None] * nrm, 0.0)).reshape(1, 1)
    vals = jnp.concatenate([p0, p1, p2, jnp.zeros((1, 125), jnp.float32)],
                           axis=1)

    @pl.when(i == 0)
    def _():
        sums_ref[...] = jnp.zeros((1, 128), jnp.float32)

    sums_ref[...] += vals


def _t2(a2a, a2b, rs, W2, b2, c1t, c2t):
    return pl.pallas_call(
        _t2_body,
        grid=(10,),
        in_specs=[
            pl.BlockSpec((2, R, 128), lambda i: (0, i, 0)),
            pl.BlockSpec((2, R, 128), lambda i: (0, i, 0)),
            pl.BlockSpec((2, R), lambda i: (0, i)),
            pl.BlockSpec((256, 256), lambda i: (0, 0)),
            pl.BlockSpec((1, 256), lambda i: (0, 0)),
            pl.BlockSpec((256, 512), lambda i: (0, 0)),
            pl.BlockSpec((256, 1024), lambda i: (0, 0)),
        ],
        out_specs=[
            pl.BlockSpec((1, 1, R), lambda i: (i, 0, 0)),
            pl.BlockSpec((1, 1, R), lambda i: (i, 0, 0)),
            pl.BlockSpec((1, 128), lambda i: (0, 0)),
        ],
        out_shape=[jax.ShapeDtypeStruct((10, 1, R), jnp.int32),
                   jax.ShapeDtypeStruct((10, 1, R), jnp.int32),
                   jax.ShapeDtypeStruct((1, 128), jnp.float32)],
    )(a2a, a2b, rs, W2, b2, c1t, c2t)


def _pad_rows(x):
    return jnp.concatenate(
        [x, jnp.zeros((NPAD - N,) + x.shape[1:], x.dtype)], axis=0)


def kernel(feats, edge_index, W1, b1, W2, b2, cb1, cb2,
           p1w1, p1b1, p1w2, p1b2, p2w1, p2b1, p2w2, p2b2):
    srcr = edge_index[0].reshape(NW, NCHUNK, CHUNK)
    dstr = edge_index[1].reshape(NW, NCHUNK, CHUNK)
    zeros1 = jnp.zeros((NPAD,), jnp.float32)
    zeros2 = jnp.zeros((NPAD, 128), jnp.float32)

    degp = _deg_kernel()(srcr, dstr, zeros1)
    xs, rs = _t0(degp, _pad_rows(feats))

    agg1p = _agg_kernel()(xs, srcr, dstr, zeros2)
    h1a, h1b = _t1(agg1p, rs, W1, b1.reshape(1, 256))

    a2a = _agg_kernel()(h1a, srcr, dstr, zeros2)
    a2b = _agg_kernel()(h1b, srcr, dstr, zeros2)

    cbn1, tab1 = _knorm(cb1, p1w1, p1b1.reshape(1, 128),
                        p1w2, p1b2.reshape(1, 64))
    cbn2, tab2 = _knorm(cb2, p2w1, p2b1.reshape(1, 128),
                        p2w2, p2b2.reshape(1, 64))

    idx1, idx2, sums = _t2(a2a, a2b, rs, W2, b2.reshape(1, 256),
                           cbn1.T, cbn2.T)
    i1r = idx1.reshape(NW, 4, CHUNK)
    i2r = idx2.reshape(NW, 4, CHUNK)

    o1, o2 = _proj_kernel()(tab1, tab2, i1r, i2r)
    proj1 = o1[:N, :64]
    proj2 = o2[:N, :64]

    scale = 0.25 / (N * 256)
    c1 = scale * (N - 2.0 * sums[0, 1] + sums[0, 0])
    c2 = scale * (N - 2.0 * sums[0, 2] + sums[0, 0])
    return (proj1, proj2, c1, c2)
